# baseline (speedup 1.0000x reference)
import functools

import jax
import jax.numpy as jnp
from jax import lax
from jax.experimental import pallas as pl
from jax.experimental.pallas import tpu as pltpu
from jax.experimental.pallas import tpu_sc as plsc

_VOCAB = 1_000_000
_DIM = 32
_BATCH = 16384

_NC = 2
_NS = 16
_L = 16
_NW = _NC * _NS
_BPW = _BATCH // _NW
_CHUNK = 128
_NCH = _BPW // _CHUNK
_GROUPS = _BPW // _L

_mesh = plsc.VectorSubcoreMesh(core_axis_name="c", subcore_axis_name="s")


@functools.partial(
    pl.kernel,
    mesh=_mesh,
    out_type=jax.ShapeDtypeStruct((_BATCH,), jnp.float32),
    compiler_params=pltpu.CompilerParams(needs_layout_passes=False,
                                         use_tc_tiling_on_sc=False),
    scratch_types=[
        pltpu.VMEM((_NCH, _CHUNK), jnp.int32),
        pltpu.VMEM((_NCH, _CHUNK), jnp.int32),
        pltpu.VMEM((_BPW, _DIM), jnp.float32),
        pltpu.VMEM((_BPW, _DIM), jnp.float32),
        pltpu.VMEM((_BPW,), jnp.float32),
        pltpu.SemaphoreType.DMA,
        pltpu.SemaphoreType.DMA,
    ],
)
def _w2v_kernel(w1_hbm, w2_hbm, emb_hbm, out_hbm,
                idx1_v, idx2_v, r1_v, r2_v, out_v, sem1, sem2):
    wid = lax.axis_index("s") * _NC + lax.axis_index("c")
    base = wid * _BPW

    pltpu.sync_copy(w1_hbm.at[wid], idx1_v)
    pltpu.sync_copy(w2_hbm.at[wid], idx2_v)

    copies = []
    for j in range(_NCH):
        dst = pl.ds(j * _CHUNK, _CHUNK)
        copies.append(pltpu.async_copy(emb_hbm.at[idx1_v.at[j]],
                                       r1_v.at[dst], sem1))
        copies.append(pltpu.async_copy(emb_hbm.at[idx2_v.at[j]],
                                       r2_v.at[dst], sem2))
    for c in copies:
        c.wait()

    iota16 = lax.iota(jnp.int32, _L)

    def group_body(g, carry):
        rows = g * _L + iota16
        acc = jnp.zeros((_L,), jnp.float32)
        for d in range(_DIM):
            cols = jnp.full((_L,), d, jnp.int32)
            a = plsc.load_gather(r1_v, [rows, cols])
            b = plsc.load_gather(r2_v, [rows, cols])
            acc = acc + a * b
        out_v[pl.ds(g * _L, _L)] = 1.0 / (1.0 + jnp.exp(-acc))
        return carry

    lax.fori_loop(0, _GROUPS, group_body, 0)

    pltpu.sync_copy(out_v, out_hbm.at[pl.ds(base, _BPW)])


def kernel(word1, word2, emb):
    w1 = word1.astype(jnp.int32).reshape(_NW, _NCH, _CHUNK)
    w2 = word2.astype(jnp.int32).reshape(_NW, _NCH, _CHUNK)
    return _w2v_kernel(w1, w2, emb)
